# Initial kernel scaffold; baseline (speedup 1.0000x reference)
import jax
import jax.numpy as jnp
from jax.experimental import pallas as pl


def kernel(x1, x2):
    raise NotImplementedError("write your pallas kernel here")


# speedup vs baseline: 5.2527x; 5.2527x over previous
import jax
import jax.numpy as jnp
from jax.experimental import pallas as pl


def _assemble_body(x1_ref, x2_ref, o_ref):
    a = x1_ref[...]
    b = x2_ref[...]
    k, nb, f = a.shape
    a_b = jnp.broadcast_to(a[:, None, :, :], (k, nb, nb, f))
    b_b = jnp.broadcast_to(b[:, :, None, :], (k, nb, nb, f))
    o_ref[...] = jnp.concatenate([a_b, b_b], axis=-1)


def kernel(x1, x2):
    nbatch, nk, nb, f = x1.shape
    rows = nbatch * nk
    x1f = x1.reshape(rows, nb, f)
    x2f = x2.reshape(rows, nb, f)
    blk = 128
    out = pl.pallas_call(
        _assemble_body,
        grid=(rows // blk,),
        in_specs=[
            pl.BlockSpec((blk, nb, f), lambda i: (i, 0, 0)),
            pl.BlockSpec((blk, nb, f), lambda i: (i, 0, 0)),
        ],
        out_specs=pl.BlockSpec((blk, nb, nb, 2 * f), lambda i: (i, 0, 0, 0)),
        out_shape=jax.ShapeDtypeStruct((rows, nb, nb, 2 * f), jnp.float32),
    )(x1f, x2f)
    return out.reshape(nbatch, nk, nb, nb, 2 * f)
